# baseline (speedup 1.0000x reference)
import functools
import math

import jax
import jax.numpy as jnp
from jax import lax
from jax.experimental import pallas as pl
from jax.experimental.pallas import tpu as pltpu

B = 16384
D = 128
HALF = D // 2
BR = 512


def _body(idx_ref, w_ref, out_ref):
    idxf = idx_ref[...].astype(jnp.float32)
    w = w_ref[...]
    args = idxf * w
    s = jnp.sin(args)
    c = jnp.cos(args)
    lane = lax.broadcasted_iota(jnp.int32, (BR, D), 1)
    out_ref[...] = jnp.where(lane < HALF, s, c)


@functools.lru_cache(maxsize=None)
def _make():
    grid = (B // BR,)
    return pl.pallas_call(
        _body,
        grid=grid,
        in_specs=[
            pl.BlockSpec((BR, 1), lambda i: (i, 0)),
            pl.BlockSpec((1, D), lambda i: (0, 0)),
        ],
        out_specs=pl.BlockSpec((BR, D), lambda i: (i, 0)),
        out_shape=jax.ShapeDtypeStruct((B, D), jnp.float32),
    )


def kernel(idx, embedding):
    emb = math.log(10000.0) / (HALF - 1)
    w = jnp.exp(jnp.arange(HALF, dtype=jnp.float32) * -emb)
    wfull = jnp.concatenate([w, w]).reshape(1, D)
    return _make()(idx.astype(jnp.int32).reshape(B, 1), wfull)


# speedup vs baseline: 1.6126x; 1.6126x over previous
import functools

import jax
import jax.numpy as jnp
from jax import lax
from jax.experimental import pallas as pl
from jax.experimental.pallas import tpu as pltpu
from jax.experimental.pallas import tpu_sc as plsc

B = 16384
D = 128
CH = 128


@functools.lru_cache(maxsize=None)
def _make_gather():
    info = plsc.get_sparse_core_info()
    nc, ns = info.num_cores, info.num_subcores
    nw = nc * ns
    b_per_w = B // nw
    n_ch = b_per_w // CH
    mesh = plsc.VectorSubcoreMesh(core_axis_name="c", subcore_axis_name="s")

    @functools.partial(
        pl.kernel,
        mesh=mesh,
        out_type=jax.ShapeDtypeStruct((B, D), jnp.float32),
        scratch_types=[
            pltpu.VMEM((b_per_w,), jnp.int32),
            pltpu.VMEM((b_per_w, D), jnp.float32),
            pltpu.SemaphoreType.DMA,
            pltpu.SemaphoreType.DMA,
        ],
    )
    def k(table_hbm, idx_hbm, out_hbm, idx_v, rows_v, gsem, osem):
        wid = lax.axis_index("s") * nc + lax.axis_index("c")
        base = wid * b_per_w
        pltpu.sync_copy(idx_hbm.at[pl.ds(base, b_per_w)], idx_v)
        pltpu.async_copy(table_hbm.at[idx_v], rows_v, gsem).wait()
        pltpu.async_copy(rows_v, out_hbm.at[pl.ds(base, b_per_w)], osem).wait()

    return k


def kernel(idx, embedding):
    k = _make_gather()
    return k(embedding, idx.astype(jnp.int32))
